# baseline (speedup 1.0000x reference)
import jax
import jax.numpy as jnp
from jax.experimental import pallas as pl

_MB = 4096


def _head_kernel(x_ref, w_ref, b_ref, o_ref):
    X = x_ref[0]
    W = w_ref[...]
    Y = jnp.dot(W, X, preferred_element_type=jnp.float32) + b_ref[...]
    for a in range(3):
        o_ref[0, a] = Y[6 * a:6 * a + 6].T


def kernel(x, Wc, bc):
    B, C, H, W = x.shape
    HW = H * W
    x2 = x.reshape(B, C, HW)
    out = pl.pallas_call(
        _head_kernel,
        grid=(B, HW // _MB),
        in_specs=[
            pl.BlockSpec((1, C, _MB), lambda b, m: (b, 0, m)),
            pl.BlockSpec((18, C), lambda b, m: (0, 0)),
            pl.BlockSpec((18, 1), lambda b, m: (0, 0)),
        ],
        out_specs=pl.BlockSpec((1, 3, _MB, 6), lambda b, m: (b, 0, m, 0)),
        out_shape=jax.ShapeDtypeStruct((B, 3, HW, 6), jnp.float32),
    )(x2, Wc, bc.reshape(18, 1))
    return out.reshape(B, 3, H, W, 6)


# speedup vs baseline: 3.0170x; 3.0170x over previous
import jax
import jax.numpy as jnp
from jax.experimental import pallas as pl

_MB = 8192


def _head_kernel(x_ref, w_ref, b_ref, o_ref):
    X = x_ref[0]
    W = w_ref[...]
    o_ref[0] = jnp.dot(W, X, preferred_element_type=jnp.float32) + b_ref[...]


def kernel(x, Wc, bc):
    B, C, H, W = x.shape
    HW = H * W
    x2 = x.reshape(B, C, HW)
    out = pl.pallas_call(
        _head_kernel,
        grid=(B, HW // _MB),
        in_specs=[
            pl.BlockSpec((1, C, _MB), lambda b, m: (b, 0, m)),
            pl.BlockSpec((18, C), lambda b, m: (0, 0)),
            pl.BlockSpec((18, 1), lambda b, m: (0, 0)),
        ],
        out_specs=pl.BlockSpec((1, 18, _MB), lambda b, m: (b, 0, m)),
        out_shape=jax.ShapeDtypeStruct((B, 18, HW), jnp.float32),
    )(x2, Wc, bc.reshape(18, 1))
    return jnp.transpose(out.reshape(B, 3, 6, H, W), (0, 1, 3, 4, 2))


# speedup vs baseline: 4.3599x; 1.4451x over previous
import jax
import jax.numpy as jnp
from jax.experimental import pallas as pl

_MB = 4096


def _head_kernel(x_ref, w_ref, b_ref, o_ref):
    o_ref[...] = (
        jnp.dot(w_ref[...], x_ref[...], preferred_element_type=jnp.float32)
        + b_ref[...]
    )


def kernel(x, Wc, bc):
    B, C, H, W = x.shape
    HW = H * W
    x2 = x.reshape(B * C, HW)
    Wbig = jnp.kron(jnp.eye(B, dtype=Wc.dtype), Wc)
    bbig = jnp.tile(bc, B).reshape(B * 18, 1)
    out = pl.pallas_call(
        _head_kernel,
        grid=(HW // _MB,),
        in_specs=[
            pl.BlockSpec((B * C, _MB), lambda m: (0, m)),
            pl.BlockSpec((B * 18, B * C), lambda m: (0, 0)),
            pl.BlockSpec((B * 18, 1), lambda m: (0, 0)),
        ],
        out_specs=pl.BlockSpec((B * 18, _MB), lambda m: (0, m)),
        out_shape=jax.ShapeDtypeStruct((B * 18, HW), jnp.float32),
    )(x2, Wbig, bbig)
    return jnp.transpose(out.reshape(B, 3, 6, H, W), (0, 1, 3, 4, 2))


# speedup vs baseline: 13.4337x; 3.0812x over previous
import jax
import jax.numpy as jnp
from jax.experimental import pallas as pl

_HB = 64


def _head_kernel(x_ref, w_ref, b_ref, o_ref):
    X = x_ref[0]
    W = w_ref[...]
    o_ref[0] = (
        jax.lax.dot_general(W, X, (((1,), (0,)), ((), ())),
                            preferred_element_type=jnp.float32)
        + b_ref[...]
    )


def kernel(x, Wc, bc):
    B, C, H, W = x.shape
    out = pl.pallas_call(
        _head_kernel,
        grid=(B, H // _HB),
        in_specs=[
            pl.BlockSpec((1, C, _HB, W), lambda b, h: (b, 0, h, 0)),
            pl.BlockSpec((18, C), lambda b, h: (0, 0)),
            pl.BlockSpec((18, 1, 1), lambda b, h: (0, 0, 0)),
        ],
        out_specs=pl.BlockSpec((1, 18, _HB, W), lambda b, h: (b, 0, h, 0)),
        out_shape=jax.ShapeDtypeStruct((B, 18, H, W), jnp.float32),
    )(x, Wc, bc.reshape(18, 1, 1))
    return jnp.transpose(out.reshape(B, 3, 6, H, W), (0, 1, 3, 4, 2))


# speedup vs baseline: 15.1583x; 1.1284x over previous
import jax
import jax.numpy as jnp
from jax.experimental import pallas as pl

_HB = 128


def _head_kernel(x_ref, w_ref, b_ref, o_ref):
    X = x_ref[0]
    W = w_ref[...]
    o_ref[0] = (
        jax.lax.dot_general(W, X, (((1,), (0,)), ((), ())),
                            preferred_element_type=jnp.float32)
        + b_ref[...]
    )


def kernel(x, Wc, bc):
    B, C, H, W = x.shape
    out = pl.pallas_call(
        _head_kernel,
        grid=(B, H // _HB),
        in_specs=[
            pl.BlockSpec((1, C, _HB, W), lambda b, h: (b, 0, h, 0)),
            pl.BlockSpec((18, C), lambda b, h: (0, 0)),
            pl.BlockSpec((18, 1, 1), lambda b, h: (0, 0, 0)),
        ],
        out_specs=pl.BlockSpec((1, 18, _HB, W), lambda b, h: (b, 0, h, 0)),
        out_shape=jax.ShapeDtypeStruct((B, 18, H, W), jnp.float32),
    )(x, Wc, bc.reshape(18, 1, 1))
    return jnp.transpose(out.reshape(B, 3, 6, H, W), (0, 1, 3, 4, 2))


# speedup vs baseline: 16.0250x; 1.0572x over previous
import jax
import jax.numpy as jnp
from jax.experimental import pallas as pl

_HB = 256


def _head_kernel(x_ref, w_ref, b_ref, o_ref):
    X = x_ref[0]
    W = w_ref[...]
    o_ref[0] = (
        jax.lax.dot_general(W, X, (((1,), (0,)), ((), ())),
                            preferred_element_type=jnp.float32)
        + b_ref[...]
    )


def kernel(x, Wc, bc):
    B, C, H, W = x.shape
    out = pl.pallas_call(
        _head_kernel,
        grid=(B, H // _HB),
        in_specs=[
            pl.BlockSpec((1, C, _HB, W), lambda b, h: (b, 0, h, 0)),
            pl.BlockSpec((18, C), lambda b, h: (0, 0)),
            pl.BlockSpec((18, 1, 1), lambda b, h: (0, 0, 0)),
        ],
        out_specs=pl.BlockSpec((1, 18, _HB, W), lambda b, h: (b, 0, h, 0)),
        out_shape=jax.ShapeDtypeStruct((B, 18, H, W), jnp.float32),
    )(x, Wc, bc.reshape(18, 1, 1))
    return jnp.transpose(out.reshape(B, 3, 6, H, W), (0, 1, 3, 4, 2))
